# Initial kernel scaffold; baseline (speedup 1.0000x reference)
import jax
import jax.numpy as jnp
from jax.experimental import pallas as pl


def kernel(x, edge_index, Wl0, bl0, Wr0, g0, b0, Wl1, bl1, Wr1):
    raise NotImplementedError("write your pallas kernel here")


# speedup vs baseline: 4.1917x; 4.1917x over previous
import functools

import jax
import jax.numpy as jnp
from jax import lax
from jax.experimental import pallas as pl
from jax.experimental.pallas import tpu as pltpu
from jax.experimental.pallas import tpu_sc as plsc

_NC = 2
_NS = 16
_K = 80


def _worker_split(N):
  rps = (N // _NS) // 8 * 8
  tail0 = _NS * rps
  return rps, tail0, N - tail0


def _build_seg_sum(N, D, E):
  NW = _NC * _NS
  per_w = E // NW
  n_chunks = per_w // _K
  rps, tail0, tail = _worker_split(N)
  mesh = plsc.VectorSubcoreMesh(core_axis_name="c", subcore_axis_name="s")

  @functools.partial(
      pl.kernel, mesh=mesh,
      out_type=jax.ShapeDtypeStruct((_NC * N, D), jnp.float32),
      scratch_types=[
          pltpu.VMEM((_K,), jnp.int32),
          pltpu.VMEM((_K,), jnp.int32),
          pltpu.VMEM((_K, D), jnp.float32),
          pltpu.VMEM_SHARED((N, D), jnp.float32),
          pltpu.SemaphoreType.DMA,
      ])
  def seg_sum(x_hbm, src_hbm, dst_hbm, zeros_hbm, out_hbm,
              src_v, dst_v, rows_v, acc_sh, sem):
    cid = lax.axis_index("c")
    sid = lax.axis_index("s")
    wid = sid * _NC + cid
    r0 = sid * rps

    pltpu.sync_copy(zeros_hbm.at[pl.ds(r0, rps)], acc_sh.at[pl.ds(r0, rps)])
    if tail:
      @pl.when(sid == 0)
      def _():
        pltpu.sync_copy(zeros_hbm.at[pl.ds(tail0, tail)],
                        acc_sh.at[pl.ds(tail0, tail)])
    plsc.subcore_barrier()

    base0 = wid * per_w

    @pl.loop(0, n_chunks)
    def _(i):
      base = base0 + i * _K
      pltpu.sync_copy(src_hbm.at[pl.ds(base, _K)], src_v)
      pltpu.sync_copy(dst_hbm.at[pl.ds(base, _K)], dst_v)
      pltpu.async_copy(x_hbm.at[src_v], rows_v, sem).wait()
      pltpu.sync_copy(rows_v, acc_sh.at[dst_v], add=True)

    plsc.subcore_barrier()
    pltpu.sync_copy(acc_sh.at[pl.ds(r0, rps)],
                    out_hbm.at[pl.ds(cid * N + r0, rps)])
    if tail:
      @pl.when(sid == 0)
      def _():
        pltpu.sync_copy(acc_sh.at[pl.ds(tail0, tail)],
                        out_hbm.at[pl.ds(cid * N + tail0, tail)])

  return seg_sum


def _build_count(N, D, E):
  NW = _NC * _NS
  per_w = E // NW
  n_chunks = per_w // _K
  rps, tail0, tail = _worker_split(N)
  mesh = plsc.VectorSubcoreMesh(core_axis_name="c", subcore_axis_name="s")

  @functools.partial(
      pl.kernel, mesh=mesh,
      out_type=jax.ShapeDtypeStruct((_NC * N, D), jnp.float32),
      scratch_types=[
          pltpu.VMEM((_K,), jnp.int32),
          pltpu.VMEM((_K, D), jnp.float32),
          pltpu.VMEM_SHARED((N, D), jnp.float32),
      ])
  def count(dst_hbm, zeros_hbm, ones_hbm, out_hbm, dst_v, ones_v, cnt_sh):
    cid = lax.axis_index("c")
    sid = lax.axis_index("s")
    wid = sid * _NC + cid
    r0 = sid * rps

    pltpu.sync_copy(zeros_hbm.at[pl.ds(r0, rps)], cnt_sh.at[pl.ds(r0, rps)])
    pltpu.sync_copy(ones_hbm, ones_v)
    if tail:
      @pl.when(sid == 0)
      def _():
        pltpu.sync_copy(zeros_hbm.at[pl.ds(tail0, tail)],
                        cnt_sh.at[pl.ds(tail0, tail)])
    plsc.subcore_barrier()

    base0 = wid * per_w

    @pl.loop(0, n_chunks)
    def _(i):
      pltpu.sync_copy(dst_hbm.at[pl.ds(base0 + i * _K, _K)], dst_v)
      pltpu.sync_copy(ones_v, cnt_sh.at[dst_v], add=True)

    plsc.subcore_barrier()
    pltpu.sync_copy(cnt_sh.at[pl.ds(r0, rps)],
                    out_hbm.at[pl.ds(cid * N + r0, rps)])
    if tail:
      @pl.when(sid == 0)
      def _():
        pltpu.sync_copy(cnt_sh.at[pl.ds(tail0, tail)],
                        out_hbm.at[pl.ds(cid * N + tail0, tail)])

  return count


def _dot_t(a, w):
  return lax.dot_general(a, w, (((1,), (1,)), ((), ())),
                         precision=lax.Precision.HIGHEST,
                         preferred_element_type=jnp.float32)


_BN = 1000


def _dense_layer0(x, Sa, Sb, Ca, Cb, Wl0, bl0, Wr0):
  N, D = x.shape
  nb = N // _BN

  def body(x_ref, sa_ref, sb_ref, ca_ref, cb_ref, wl_ref, bl_ref, wr_ref,
           out_ref, st_ref):
    i = pl.program_id(0)
    S0 = sa_ref[...] + sb_ref[...]
    cnt = ca_ref[...] + cb_ref[...]
    agg = S0 / jnp.maximum(cnt, 1.0)
    h = _dot_t(agg, wl_ref[...]) + bl_ref[...] + _dot_t(x_ref[...],
                                                        wr_ref[...])
    h = jnp.maximum(h, 0.0)
    out_ref[...] = h
    st = jnp.concatenate(
        [jnp.sum(h, axis=0, keepdims=True),
         jnp.sum(h * h, axis=0, keepdims=True)], axis=0)

    @pl.when(i == 0)
    def _():
      st_ref[...] = st

    @pl.when(i > 0)
    def _():
      st_ref[...] = st_ref[...] + st

  row = pl.BlockSpec((_BN, D), lambda i: (i, 0))
  mat = pl.BlockSpec((D, D), lambda i: (0, 0))
  vec = pl.BlockSpec((1, D), lambda i: (0, 0))
  return pl.pallas_call(
      body,
      grid=(nb,),
      in_specs=[row, row, row, row, row, mat, vec, mat],
      out_specs=[row, pl.BlockSpec((2, D), lambda i: (0, 0))],
      out_shape=[jax.ShapeDtypeStruct((N, D), jnp.float32),
                 jax.ShapeDtypeStruct((2, D), jnp.float32)],
  )(x, Sa, Sb, Ca, Cb, Wl0, bl0.reshape(1, D), Wr0)


def _dense_layer1(h0, Sa, Sb, Ca, Cb, Wl1, bl1, Wr1, a, c):
  N, D = h0.shape
  nb = N // _BN

  def body(h_ref, sa_ref, sb_ref, ca_ref, cb_ref, wl_ref, bl_ref, wr_ref,
           a_ref, c_ref, out_ref):
    S1 = sa_ref[...] + sb_ref[...]
    cnt = ca_ref[...] + cb_ref[...]
    agg = S1 / jnp.maximum(cnt, 1.0)
    agg_bn = agg * a_ref[...] + c_ref[...]
    h_bn = h_ref[...] * a_ref[...] + c_ref[...]
    h = _dot_t(agg_bn, wl_ref[...]) + bl_ref[...] + _dot_t(h_bn, wr_ref[...])
    out_ref[...] = jnp.maximum(h, 0.0)

  row = pl.BlockSpec((_BN, D), lambda i: (i, 0))
  mat = pl.BlockSpec((D, D), lambda i: (0, 0))
  vec = pl.BlockSpec((1, D), lambda i: (0, 0))
  return pl.pallas_call(
      body,
      grid=(nb,),
      in_specs=[row, row, row, row, row, mat, vec, mat, vec, vec],
      out_specs=row,
      out_shape=jax.ShapeDtypeStruct((N, D), jnp.float32),
  )(h0, Sa, Sb, Ca, Cb, Wl1, bl1.reshape(1, D), Wr1, a.reshape(1, D),
    c.reshape(1, D))


def kernel(x, edge_index, Wl0, bl0, Wr0, g0, b0, Wl1, bl1, Wr1):
  N, D = x.shape
  E = edge_index.shape[1]
  src = edge_index[0]
  dst = edge_index[1]
  zeros = jnp.zeros((N, D), jnp.float32)
  ones = jnp.ones((_K, D), jnp.float32)

  cnt = _build_count(N, D, E)(dst, zeros, ones)
  Ca, Cb = cnt[:N], cnt[N:]

  seg = _build_seg_sum(N, D, E)
  S0 = seg(x, src, dst, zeros)
  h0, st = _dense_layer0(x, S0[:N], S0[N:], Ca, Cb, Wl0, bl0, Wr0)

  mu = st[0] / N
  var = st[1] / N - mu * mu
  a = g0 * lax.rsqrt(var + 1e-5)
  c = b0 - mu * a

  S1 = seg(h0, src, dst, zeros)
  return _dense_layer1(h0, S1[:N], S1[N:], Ca, Cb, Wl1, bl1, Wr1, a, c)


# speedup vs baseline: 5.8258x; 1.3898x over previous
import functools

import jax
import jax.numpy as jnp
from jax import lax
from jax.experimental import pallas as pl
from jax.experimental.pallas import tpu as pltpu
from jax.experimental.pallas import tpu_sc as plsc

_NC = 2
_NS = 16
_K = 80


def _worker_split(N):
  rps = (N // _NS) // 8 * 8
  tail0 = _NS * rps
  return rps, tail0, N - tail0


def _build_seg_sum(N, D, E):
  K = 40
  NW = _NC * _NS
  per_w = E // NW
  n_chunks = per_w // K
  rps, tail0, tail = _worker_split(N)
  mesh = plsc.VectorSubcoreMesh(core_axis_name="c", subcore_axis_name="s")

  @functools.partial(
      pl.kernel, mesh=mesh,
      out_type=jax.ShapeDtypeStruct((_NC * N, D), jnp.float32),
      scratch_types=[
          pltpu.VMEM((K,), jnp.int32),
          pltpu.VMEM((K,), jnp.int32),
          pltpu.VMEM((K,), jnp.int32),
          pltpu.VMEM((K,), jnp.int32),
          pltpu.VMEM((K, D), jnp.float32),
          pltpu.VMEM((K, D), jnp.float32),
          pltpu.VMEM_SHARED((N, D), jnp.float32),
          pltpu.SemaphoreType.DMA,
          pltpu.SemaphoreType.DMA,
          pltpu.SemaphoreType.DMA,
          pltpu.SemaphoreType.DMA,
      ])
  def seg_sum(x_hbm, src_hbm, dst_hbm, zeros_hbm, out_hbm,
              s0, s1, d0, d1, rows0, rows1, acc_sh,
              isem0, isem1, gsem0, gsem1):
    cid = lax.axis_index("c")
    sid = lax.axis_index("s")
    wid = sid * _NC + cid
    r0 = sid * rps
    base0 = wid * per_w

    def idx_start(i, sv, dv, isem):
      pltpu.async_copy(src_hbm.at[pl.ds(base0 + i * K, K)], sv, isem)
      pltpu.async_copy(dst_hbm.at[pl.ds(base0 + i * K, K)], dv, isem)

    def idx_wait(i, sv, dv, isem):
      pltpu.make_async_copy(src_hbm.at[pl.ds(base0 + i * K, K)], sv,
                            isem).wait()
      pltpu.make_async_copy(dst_hbm.at[pl.ds(base0 + i * K, K)], dv,
                            isem).wait()

    def gather_start(sv, rows, gsem):
      pltpu.async_copy(x_hbm.at[sv], rows, gsem)

    def gather_wait(sv, rows, gsem):
      pltpu.make_async_copy(x_hbm.at[sv], rows, gsem).wait()

    pltpu.sync_copy(zeros_hbm.at[pl.ds(r0, rps)], acc_sh.at[pl.ds(r0, rps)])
    if tail:
      @pl.when(sid == 0)
      def _():
        pltpu.sync_copy(zeros_hbm.at[pl.ds(tail0, tail)],
                        acc_sh.at[pl.ds(tail0, tail)])
    idx_start(0, s0, d0, isem0)
    idx_start(1, s1, d1, isem1)
    plsc.subcore_barrier()

    idx_wait(0, s0, d0, isem0)
    gather_start(s0, rows0, gsem0)

    @pl.loop(0, n_chunks, step=2)
    def _(i):
      idx_wait(i + 1, s1, d1, isem1)
      gather_start(s1, rows1, gsem1)
      gather_wait(s0, rows0, gsem0)
      pltpu.sync_copy(rows0, acc_sh.at[d0], add=True)

      @pl.when(i + 2 < n_chunks)
      def _():
        idx_start(i + 2, s0, d0, isem0)

      gather_wait(s1, rows1, gsem1)

      @pl.when(i + 2 < n_chunks)
      def _():
        idx_wait(i + 2, s0, d0, isem0)
        gather_start(s0, rows0, gsem0)

      pltpu.sync_copy(rows1, acc_sh.at[d1], add=True)

      @pl.when(i + 3 < n_chunks)
      def _():
        idx_start(i + 3, s1, d1, isem1)

    plsc.subcore_barrier()
    pltpu.sync_copy(acc_sh.at[pl.ds(r0, rps)],
                    out_hbm.at[pl.ds(cid * N + r0, rps)])
    if tail:
      @pl.when(sid == 0)
      def _():
        pltpu.sync_copy(acc_sh.at[pl.ds(tail0, tail)],
                        out_hbm.at[pl.ds(cid * N + tail0, tail)])

  return seg_sum


def _build_count(N, D, E):
  NW = _NC * _NS
  per_w = E // NW
  n_chunks = per_w // _K
  rps, tail0, tail = _worker_split(N)
  mesh = plsc.VectorSubcoreMesh(core_axis_name="c", subcore_axis_name="s")

  @functools.partial(
      pl.kernel, mesh=mesh,
      out_type=jax.ShapeDtypeStruct((_NC * N, D), jnp.float32),
      scratch_types=[
          pltpu.VMEM((_K,), jnp.int32),
          pltpu.VMEM((_K,), jnp.int32),
          pltpu.VMEM((_K, D), jnp.float32),
          pltpu.VMEM_SHARED((N, D), jnp.float32),
          pltpu.SemaphoreType.DMA,
          pltpu.SemaphoreType.DMA,
      ])
  def count(dst_hbm, zeros_hbm, ones_hbm, out_hbm,
            d0, d1, ones_v, cnt_sh, isem0, isem1):
    cid = lax.axis_index("c")
    sid = lax.axis_index("s")
    wid = sid * _NC + cid
    r0 = sid * rps
    base0 = wid * per_w

    def idx_start(i, dv, isem):
      pltpu.async_copy(dst_hbm.at[pl.ds(base0 + i * _K, _K)], dv, isem)

    def idx_wait(i, dv, isem):
      pltpu.make_async_copy(dst_hbm.at[pl.ds(base0 + i * _K, _K)], dv,
                            isem).wait()

    pltpu.sync_copy(zeros_hbm.at[pl.ds(r0, rps)], cnt_sh.at[pl.ds(r0, rps)])
    pltpu.sync_copy(ones_hbm, ones_v)
    if tail:
      @pl.when(sid == 0)
      def _():
        pltpu.sync_copy(zeros_hbm.at[pl.ds(tail0, tail)],
                        cnt_sh.at[pl.ds(tail0, tail)])
    idx_start(0, d0, isem0)
    plsc.subcore_barrier()

    @pl.loop(0, n_chunks, step=2)
    def _(i):
      idx_wait(i, d0, isem0)

      @pl.when(i + 1 < n_chunks)
      def _():
        idx_start(i + 1, d1, isem1)

      pltpu.sync_copy(ones_v, cnt_sh.at[d0], add=True)

      @pl.when(i + 2 < n_chunks)
      def _():
        idx_start(i + 2, d0, isem0)

      @pl.when(i + 1 < n_chunks)
      def _():
        idx_wait(i + 1, d1, isem1)
        pltpu.sync_copy(ones_v, cnt_sh.at[d1], add=True)

    plsc.subcore_barrier()
    pltpu.sync_copy(cnt_sh.at[pl.ds(r0, rps)],
                    out_hbm.at[pl.ds(cid * N + r0, rps)])
    if tail:
      @pl.when(sid == 0)
      def _():
        pltpu.sync_copy(cnt_sh.at[pl.ds(tail0, tail)],
                        out_hbm.at[pl.ds(cid * N + tail0, tail)])

  return count


def _dot_t(a, w):
  return lax.dot_general(a, w, (((1,), (1,)), ((), ())),
                         precision=lax.Precision.HIGHEST,
                         preferred_element_type=jnp.float32)


_BN = 1000


def _dense_layer0(x, Sa, Sb, Ca, Cb, Wl0, bl0, Wr0):
  N, D = x.shape
  nb = N // _BN

  def body(x_ref, sa_ref, sb_ref, ca_ref, cb_ref, wl_ref, bl_ref, wr_ref,
           out_ref, st_ref):
    i = pl.program_id(0)
    S0 = sa_ref[...] + sb_ref[...]
    cnt = ca_ref[...] + cb_ref[...]
    agg = S0 / jnp.maximum(cnt, 1.0)
    h = _dot_t(agg, wl_ref[...]) + bl_ref[...] + _dot_t(x_ref[...],
                                                        wr_ref[...])
    h = jnp.maximum(h, 0.0)
    out_ref[...] = h
    st = jnp.concatenate(
        [jnp.sum(h, axis=0, keepdims=True),
         jnp.sum(h * h, axis=0, keepdims=True)], axis=0)

    @pl.when(i == 0)
    def _():
      st_ref[...] = st

    @pl.when(i > 0)
    def _():
      st_ref[...] = st_ref[...] + st

  row = pl.BlockSpec((_BN, D), lambda i: (i, 0))
  mat = pl.BlockSpec((D, D), lambda i: (0, 0))
  vec = pl.BlockSpec((1, D), lambda i: (0, 0))
  return pl.pallas_call(
      body,
      grid=(nb,),
      in_specs=[row, row, row, row, row, mat, vec, mat],
      out_specs=[row, pl.BlockSpec((2, D), lambda i: (0, 0))],
      out_shape=[jax.ShapeDtypeStruct((N, D), jnp.float32),
                 jax.ShapeDtypeStruct((2, D), jnp.float32)],
  )(x, Sa, Sb, Ca, Cb, Wl0, bl0.reshape(1, D), Wr0)


def _dense_layer1(h0, Sa, Sb, Ca, Cb, Wl1, bl1, Wr1, a, c):
  N, D = h0.shape
  nb = N // _BN

  def body(h_ref, sa_ref, sb_ref, ca_ref, cb_ref, wl_ref, bl_ref, wr_ref,
           a_ref, c_ref, out_ref):
    S1 = sa_ref[...] + sb_ref[...]
    cnt = ca_ref[...] + cb_ref[...]
    agg = S1 / jnp.maximum(cnt, 1.0)
    agg_bn = agg * a_ref[...] + c_ref[...]
    h_bn = h_ref[...] * a_ref[...] + c_ref[...]
    h = _dot_t(agg_bn, wl_ref[...]) + bl_ref[...] + _dot_t(h_bn, wr_ref[...])
    out_ref[...] = jnp.maximum(h, 0.0)

  row = pl.BlockSpec((_BN, D), lambda i: (i, 0))
  mat = pl.BlockSpec((D, D), lambda i: (0, 0))
  vec = pl.BlockSpec((1, D), lambda i: (0, 0))
  return pl.pallas_call(
      body,
      grid=(nb,),
      in_specs=[row, row, row, row, row, mat, vec, mat, vec, vec],
      out_specs=row,
      out_shape=jax.ShapeDtypeStruct((N, D), jnp.float32),
  )(h0, Sa, Sb, Ca, Cb, Wl1, bl1.reshape(1, D), Wr1, a.reshape(1, D),
    c.reshape(1, D))


def kernel(x, edge_index, Wl0, bl0, Wr0, g0, b0, Wl1, bl1, Wr1):
  N, D = x.shape
  E = edge_index.shape[1]
  src = edge_index[0]
  dst = edge_index[1]
  zeros = jnp.zeros((N, D), jnp.float32)
  ones = jnp.ones((_K, D), jnp.float32)

  cnt = _build_count(N, D, E)(dst, zeros, ones)
  Ca, Cb = cnt[:N], cnt[N:]

  seg = _build_seg_sum(N, D, E)
  S0 = seg(x, src, dst, zeros)
  h0, st = _dense_layer0(x, S0[:N], S0[N:], Ca, Cb, Wl0, bl0, Wr0)

  mu = st[0] / N
  var = st[1] / N - mu * mu
  a = g0 * lax.rsqrt(var + 1e-5)
  c = b0 - mu * a

  S1 = seg(h0, src, dst, zeros)
  return _dense_layer1(h0, S1[:N], S1[N:], Ca, Cb, Wl1, bl1, Wr1, a, c)


# speedup vs baseline: 7.5473x; 1.2955x over previous
import functools

import jax
import jax.numpy as jnp
from jax import lax
from jax.experimental import pallas as pl
from jax.experimental.pallas import tpu as pltpu
from jax.experimental.pallas import tpu_sc as plsc

_NC = 2
_NS = 16
_K = 80


def _worker_split(N):
  rps = (N // _NS) // 8 * 8
  tail0 = _NS * rps
  return rps, tail0, N - tail0


def _build_seg_sum(N, D, E):
  K = 80
  NW = _NC * _NS
  per_w = E // NW
  n_chunks = per_w // K
  rps, tail0, tail = _worker_split(N)
  mesh = plsc.VectorSubcoreMesh(core_axis_name="c", subcore_axis_name="s")

  @functools.partial(
      pl.kernel, mesh=mesh,
      out_type=jax.ShapeDtypeStruct((_NC * N, D), jnp.float32),
      scratch_types=[
          pltpu.VMEM((K,), jnp.int32),
          pltpu.VMEM((K,), jnp.int32),
          pltpu.VMEM((K,), jnp.int32),
          pltpu.VMEM((K,), jnp.int32),
          pltpu.VMEM((K, D), jnp.float32),
          pltpu.VMEM((K, D), jnp.float32),
          pltpu.VMEM_SHARED((N, D), jnp.float32),
          pltpu.SemaphoreType.DMA,
          pltpu.SemaphoreType.DMA,
          pltpu.SemaphoreType.DMA,
          pltpu.SemaphoreType.DMA,
      ])
  def seg_sum(x_hbm, src_hbm, dst_hbm, zeros_hbm, out_hbm,
              s0, s1, d0, d1, rows0, rows1, acc_sh,
              isem0, isem1, gsem0, gsem1):
    cid = lax.axis_index("c")
    sid = lax.axis_index("s")
    wid = sid * _NC + cid
    r0 = sid * rps
    base0 = wid * per_w

    def idx_start(i, sv, dv, isem):
      pltpu.async_copy(src_hbm.at[pl.ds(base0 + i * K, K)], sv, isem)
      pltpu.async_copy(dst_hbm.at[pl.ds(base0 + i * K, K)], dv, isem)

    def idx_wait(i, sv, dv, isem):
      pltpu.make_async_copy(src_hbm.at[pl.ds(base0 + i * K, K)], sv,
                            isem).wait()
      pltpu.make_async_copy(dst_hbm.at[pl.ds(base0 + i * K, K)], dv,
                            isem).wait()

    def gather_start(sv, rows, gsem):
      pltpu.async_copy(x_hbm.at[sv], rows, gsem)

    def gather_wait(sv, rows, gsem):
      pltpu.make_async_copy(x_hbm.at[sv], rows, gsem).wait()

    pltpu.sync_copy(zeros_hbm.at[pl.ds(r0, rps)], acc_sh.at[pl.ds(r0, rps)])
    if tail:
      @pl.when(sid == 0)
      def _():
        pltpu.sync_copy(zeros_hbm.at[pl.ds(tail0, tail)],
                        acc_sh.at[pl.ds(tail0, tail)])
    idx_start(0, s0, d0, isem0)
    idx_start(1, s1, d1, isem1)
    plsc.subcore_barrier()

    idx_wait(0, s0, d0, isem0)
    gather_start(s0, rows0, gsem0)

    @pl.loop(0, n_chunks, step=2)
    def _(i):
      @pl.when(i + 1 < n_chunks)
      def _():
        idx_wait(i + 1, s1, d1, isem1)
        gather_start(s1, rows1, gsem1)

      gather_wait(s0, rows0, gsem0)
      pltpu.sync_copy(rows0, acc_sh.at[d0], add=True)

      @pl.when(i + 2 < n_chunks)
      def _():
        idx_start(i + 2, s0, d0, isem0)

      @pl.when(i + 1 < n_chunks)
      def _():
        gather_wait(s1, rows1, gsem1)

      @pl.when(i + 2 < n_chunks)
      def _():
        idx_wait(i + 2, s0, d0, isem0)
        gather_start(s0, rows0, gsem0)

      @pl.when(i + 1 < n_chunks)
      def _():
        pltpu.sync_copy(rows1, acc_sh.at[d1], add=True)

      @pl.when(i + 3 < n_chunks)
      def _():
        idx_start(i + 3, s1, d1, isem1)

    plsc.subcore_barrier()
    pltpu.sync_copy(acc_sh.at[pl.ds(r0, rps)],
                    out_hbm.at[pl.ds(cid * N + r0, rps)])
    if tail:
      @pl.when(sid == 0)
      def _():
        pltpu.sync_copy(acc_sh.at[pl.ds(tail0, tail)],
                        out_hbm.at[pl.ds(cid * N + tail0, tail)])

  return seg_sum


def _build_count(N, D, E):
  NW = _NC * _NS
  per_w = E // NW
  n_chunks = per_w // _K
  rps, tail0, tail = _worker_split(N)
  mesh = plsc.VectorSubcoreMesh(core_axis_name="c", subcore_axis_name="s")

  @functools.partial(
      pl.kernel, mesh=mesh,
      out_type=jax.ShapeDtypeStruct((_NC * N, D), jnp.float32),
      scratch_types=[
          pltpu.VMEM((_K,), jnp.int32),
          pltpu.VMEM((_K,), jnp.int32),
          pltpu.VMEM((_K, D), jnp.float32),
          pltpu.VMEM_SHARED((N, D), jnp.float32),
          pltpu.SemaphoreType.DMA,
          pltpu.SemaphoreType.DMA,
      ])
  def count(dst_hbm, zeros_hbm, ones_hbm, out_hbm,
            d0, d1, ones_v, cnt_sh, isem0, isem1):
    cid = lax.axis_index("c")
    sid = lax.axis_index("s")
    wid = sid * _NC + cid
    r0 = sid * rps
    base0 = wid * per_w

    def idx_start(i, dv, isem):
      pltpu.async_copy(dst_hbm.at[pl.ds(base0 + i * _K, _K)], dv, isem)

    def idx_wait(i, dv, isem):
      pltpu.make_async_copy(dst_hbm.at[pl.ds(base0 + i * _K, _K)], dv,
                            isem).wait()

    pltpu.sync_copy(zeros_hbm.at[pl.ds(r0, rps)], cnt_sh.at[pl.ds(r0, rps)])
    pltpu.sync_copy(ones_hbm, ones_v)
    if tail:
      @pl.when(sid == 0)
      def _():
        pltpu.sync_copy(zeros_hbm.at[pl.ds(tail0, tail)],
                        cnt_sh.at[pl.ds(tail0, tail)])
    idx_start(0, d0, isem0)
    plsc.subcore_barrier()

    @pl.loop(0, n_chunks, step=2)
    def _(i):
      idx_wait(i, d0, isem0)

      @pl.when(i + 1 < n_chunks)
      def _():
        idx_start(i + 1, d1, isem1)

      pltpu.sync_copy(ones_v, cnt_sh.at[d0], add=True)

      @pl.when(i + 2 < n_chunks)
      def _():
        idx_start(i + 2, d0, isem0)

      @pl.when(i + 1 < n_chunks)
      def _():
        idx_wait(i + 1, d1, isem1)
        pltpu.sync_copy(ones_v, cnt_sh.at[d1], add=True)

    plsc.subcore_barrier()
    pltpu.sync_copy(cnt_sh.at[pl.ds(r0, rps)],
                    out_hbm.at[pl.ds(cid * N + r0, rps)])
    if tail:
      @pl.when(sid == 0)
      def _():
        pltpu.sync_copy(cnt_sh.at[pl.ds(tail0, tail)],
                        out_hbm.at[pl.ds(cid * N + tail0, tail)])

  return count


def _dot_t(a, w):
  return lax.dot_general(a, w, (((1,), (1,)), ((), ())),
                         precision=lax.Precision.HIGHEST,
                         preferred_element_type=jnp.float32)


_BN = 1000


def _dense_layer0(x, Sa, Sb, Ca, Cb, Wl0, bl0, Wr0):
  N, D = x.shape
  nb = N // _BN

  def body(x_ref, sa_ref, sb_ref, ca_ref, cb_ref, wl_ref, bl_ref, wr_ref,
           out_ref, st_ref):
    i = pl.program_id(0)
    S0 = sa_ref[...] + sb_ref[...]
    cnt = ca_ref[...] + cb_ref[...]
    agg = S0 / jnp.maximum(cnt, 1.0)
    h = _dot_t(agg, wl_ref[...]) + bl_ref[...] + _dot_t(x_ref[...],
                                                        wr_ref[...])
    h = jnp.maximum(h, 0.0)
    out_ref[...] = h
    st = jnp.concatenate(
        [jnp.sum(h, axis=0, keepdims=True),
         jnp.sum(h * h, axis=0, keepdims=True)], axis=0)

    @pl.when(i == 0)
    def _():
      st_ref[...] = st

    @pl.when(i > 0)
    def _():
      st_ref[...] = st_ref[...] + st

  row = pl.BlockSpec((_BN, D), lambda i: (i, 0))
  mat = pl.BlockSpec((D, D), lambda i: (0, 0))
  vec = pl.BlockSpec((1, D), lambda i: (0, 0))
  return pl.pallas_call(
      body,
      grid=(nb,),
      in_specs=[row, row, row, row, row, mat, vec, mat],
      out_specs=[row, pl.BlockSpec((2, D), lambda i: (0, 0))],
      out_shape=[jax.ShapeDtypeStruct((N, D), jnp.float32),
                 jax.ShapeDtypeStruct((2, D), jnp.float32)],
  )(x, Sa, Sb, Ca, Cb, Wl0, bl0.reshape(1, D), Wr0)


def _dense_layer1(h0, Sa, Sb, Ca, Cb, Wl1, bl1, Wr1, a, c):
  N, D = h0.shape
  nb = N // _BN

  def body(h_ref, sa_ref, sb_ref, ca_ref, cb_ref, wl_ref, bl_ref, wr_ref,
           a_ref, c_ref, out_ref):
    S1 = sa_ref[...] + sb_ref[...]
    cnt = ca_ref[...] + cb_ref[...]
    agg = S1 / jnp.maximum(cnt, 1.0)
    agg_bn = agg * a_ref[...] + c_ref[...]
    h_bn = h_ref[...] * a_ref[...] + c_ref[...]
    h = _dot_t(agg_bn, wl_ref[...]) + bl_ref[...] + _dot_t(h_bn, wr_ref[...])
    out_ref[...] = jnp.maximum(h, 0.0)

  row = pl.BlockSpec((_BN, D), lambda i: (i, 0))
  mat = pl.BlockSpec((D, D), lambda i: (0, 0))
  vec = pl.BlockSpec((1, D), lambda i: (0, 0))
  return pl.pallas_call(
      body,
      grid=(nb,),
      in_specs=[row, row, row, row, row, mat, vec, mat, vec, vec],
      out_specs=row,
      out_shape=jax.ShapeDtypeStruct((N, D), jnp.float32),
  )(h0, Sa, Sb, Ca, Cb, Wl1, bl1.reshape(1, D), Wr1, a.reshape(1, D),
    c.reshape(1, D))


def kernel(x, edge_index, Wl0, bl0, Wr0, g0, b0, Wl1, bl1, Wr1):
  N, D = x.shape
  E = edge_index.shape[1]
  src = edge_index[0]
  dst = edge_index[1]
  zeros = jnp.zeros((N, D), jnp.float32)
  ones = jnp.ones((_K, D), jnp.float32)

  cnt = _build_count(N, D, E)(dst, zeros, ones)
  Ca, Cb = cnt[:N], cnt[N:]

  seg = _build_seg_sum(N, D, E)
  S0 = seg(x, src, dst, zeros)
  h0, st = _dense_layer0(x, S0[:N], S0[N:], Ca, Cb, Wl0, bl0, Wr0)

  mu = st[0] / N
  var = st[1] / N - mu * mu
  a = g0 * lax.rsqrt(var + 1e-5)
  c = b0 - mu * a

  S1 = seg(h0, src, dst, zeros)
  return _dense_layer1(h0, S1[:N], S1[N:], Ca, Cb, Wl1, bl1, Wr1, a, c)
